# baseline (speedup 1.0000x reference)
import functools

import jax
import jax.numpy as jnp
from jax import lax
from jax.experimental import pallas as pl
from jax.experimental.pallas import tpu as pltpu
from jax.experimental.pallas import tpu_sc as plsc

_NPOINTS = 2048
_NSAMPLE = 32


def _sc_gather_tiled(w, idx2d, b_dim, s_dim):
    mesh = plsc.VectorSubcoreMesh(core_axis_name="c", subcore_axis_name="s")
    info = plsc.get_sparse_core_info()

    sq = 8
    nq = s_dim // sq

    @functools.partial(
        pl.kernel,
        out_type=jax.ShapeDtypeStruct((b_dim, s_dim, 2, 16, 8, 128), w.dtype),
        mesh=mesh,
        scratch_types=[
            pltpu.VMEM((128, 4, 128), jnp.float32),
            pltpu.VMEM((16 * sq, 128), jnp.int32),
            pltpu.VMEM((2, 16, 4, 128), jnp.float32),
            pltpu.VMEM((2, 16, 4, 128), jnp.float32),
            pltpu.SemaphoreType.DMA,
            pltpu.SemaphoreType.DMA,
            pltpu.SemaphoreType.DMA,
            pltpu.SemaphoreType.DMA,
        ],
        compiler_params=pltpu.CompilerParams(
            use_tc_tiling_on_sc=False, needs_layout_passes=False),
    )
    def gather_kernel(w_hbm, idx_hbm, out_hbm, table_v, idx_v,
                      out_v0, out_v1, st, si, so0, so1):
        wid = lax.axis_index("s") * info.num_cores + lax.axis_index("c")
        b = wid // 2
        ct = wid % 2
        out_bufs = (out_v0, out_v1)
        sems = (so0, so1)

        def idx_src(q):
            return idx_hbm.at[pl.ds((b * s_dim + q * sq) * 16, 16 * sq)]

        for h in range(2):
            tcp = pltpu.async_copy(
                w_hbm.at[b, ct, :, pl.ds(4 * h, 4), :], table_v, st)
            icp = pltpu.async_copy(idx_src(0), idx_v, si)
            tcp.wait()
            icp.wait()

            def chunk(q, carry0, h=h):
                @pl.when(q > 0)
                def _():
                    pltpu.sync_copy(idx_src(q), idx_v)

                def pair(t, carry, q=q, h=h):
                    for p in range(2):
                        sl2 = 2 * t + p
                        s0 = q * sq + sl2 * 2

                        @pl.when(jnp.logical_or(sl2 >= 2, (q + h) > 0))
                        def _():
                            pltpu.make_async_copy(
                                out_bufs[p],
                                out_hbm.at[b, pl.ds(s0, 2), ct, :,
                                           pl.ds(4 * h, 4), :],
                                sems[p]).wait()

                        @plsc.parallel_loop(0, 32, unroll=8)
                        def ptile(pt2, p=p, sl2=sl2):
                            for gg in range(8):
                                n = idx_v[sl2 * 32 + pt2, pl.ds(gg * 16, 16)]
                                nt = lax.shift_right_logical(n, 7)
                                nj = lax.bitwise_and(n, 127)
                                so = lax.shift_right_logical(pt2, 4)
                                pt = lax.bitwise_and(pt2, 15)
                                for ci_ in range(4):
                                    ci_arr = jnp.full((16,), ci_, jnp.int32)
                                    vals = plsc.load_gather(
                                        table_v, [nt, ci_arr, nj])
                                    out_bufs[p][
                                        so, pt, ci_, pl.ds(gg * 16, 16)
                                    ] = vals
                        pltpu.async_copy(
                            out_bufs[p],
                            out_hbm.at[b, pl.ds(s0, 2), ct, :,
                                       pl.ds(4 * h, 4), :],
                            sems[p])
                    return carry

                lax.fori_loop(0, sq // 4, pair, 0)
                return carry0

            lax.fori_loop(0, nq, chunk, 0)

        for p in range(2):
            pltpu.make_async_copy(
                out_bufs[p],
                out_hbm.at[b, pl.ds(0, 2), ct, :, pl.ds(4, 4), :],
                sems[p]).wait()

    return gather_kernel(w, idx2d)


def _rotl(x, r):
    return (x << jnp.uint32(r)) | (x >> jnp.uint32(32 - r))


def _tf_rounds(x0, x1, rots):
    for r in rots:
        x0 = x0 + x1
        x1 = _rotl(x1, r)
        x1 = x1 ^ x0
    return x0, x1


def _group_indices_2d(b, n):
    key = jax.random.key(42)
    k2 = jax.random.split(key)[1]
    kd = jax.random.key_data(jax.random.split(k2, 2)[1]).astype(jnp.uint32)
    ks0, ks1 = kd[0], kd[1]
    ks2 = ks0 ^ ks1 ^ jnp.uint32(0x1BD11BDA)
    r0 = (13, 15, 26, 6)
    r1 = (17, 29, 16, 24)

    b_ = jnp.arange(b, dtype=jnp.uint32)[:, None, None, None]
    s_ = jnp.arange(_NSAMPLE, dtype=jnp.uint32)[None, :, None, None]
    pt_ = jnp.arange(16, dtype=jnp.uint32)[None, None, :, None]
    pj_ = jnp.arange(128, dtype=jnp.uint32)[None, None, None, :]
    i = (b_ * jnp.uint32(_NPOINTS * _NSAMPLE)
         + (pt_ * jnp.uint32(128) + pj_) * jnp.uint32(_NSAMPLE) + s_)
    i = i.reshape(b * _NSAMPLE * 16, 128)

    x0, x1 = jnp.uint32(0) + ks0, i + ks1
    x0, x1 = _tf_rounds(x0, x1, r0); x0 += ks1; x1 += ks2 + jnp.uint32(1)
    x0, x1 = _tf_rounds(x0, x1, r1); x0 += ks2; x1 += ks0 + jnp.uint32(2)
    x0, x1 = _tf_rounds(x0, x1, r0); x0 += ks0; x1 += ks1 + jnp.uint32(3)
    x0, x1 = _tf_rounds(x0, x1, r1); x0 += ks1; x1 += ks2 + jnp.uint32(4)
    x0, x1 = _tf_rounds(x0, x1, r0); x0 += ks2; x1 += ks0 + jnp.uint32(5)
    return ((x0 ^ x1) & jnp.uint32(n - 1)).astype(jnp.int32)


def kernel(inputs):
    b, n, c = inputs.shape
    idx2d = _group_indices_2d(b, n)
    w = inputs.transpose(0, 2, 1).reshape(b, 2, 8, 128, 128)
    w = w.transpose(0, 1, 3, 2, 4)
    o6 = _sc_gather_tiled(w, idx2d, b, _NSAMPLE)
    out = o6.transpose(0, 1, 2, 4, 3, 5).reshape(b, _NSAMPLE, c, _NPOINTS)
    return out.transpose(0, 3, 1, 2)


# speedup vs baseline: 1.1550x; 1.1550x over previous
import functools

import jax
import jax.numpy as jnp
from jax import lax
from jax.experimental import pallas as pl
from jax.experimental.pallas import tpu as pltpu
from jax.experimental.pallas import tpu_sc as plsc

_NPOINTS = 2048
_NSAMPLE = 32


def _sc_gather_tiled(w, idx2d, b_dim, s_dim):
    mesh = plsc.VectorSubcoreMesh(core_axis_name="c", subcore_axis_name="s")
    info = plsc.get_sparse_core_info()

    sq = 8
    nq = s_dim // sq

    @functools.partial(
        pl.kernel,
        out_type=jax.ShapeDtypeStruct((b_dim, s_dim, 2, 16, 8, 128), w.dtype),
        mesh=mesh,
        scratch_types=[
            pltpu.VMEM((128, 4, 128), jnp.float32),
            pltpu.VMEM((16 * sq, 128), jnp.int32),
            pltpu.VMEM((2, 16, 4, 128), jnp.float32),
            pltpu.VMEM((2, 16, 4, 128), jnp.float32),
            pltpu.SemaphoreType.DMA,
            pltpu.SemaphoreType.DMA,
            pltpu.SemaphoreType.DMA,
            pltpu.SemaphoreType.DMA,
        ],
        compiler_params=pltpu.CompilerParams(
            use_tc_tiling_on_sc=False, needs_layout_passes=False),
    )
    def gather_kernel(w_hbm, idx_hbm, out_hbm, table_v, idx_v,
                      out_v0, out_v1, st, si, so0, so1):
        wid = lax.axis_index("s") * info.num_cores + lax.axis_index("c")
        b = wid // 2
        ct = wid % 2
        out_bufs = (out_v0, out_v1)
        sems = (so0, so1)

        def idx_src(q):
            return idx_hbm.at[pl.ds((b * s_dim + q * sq) * 16, 16 * sq)]

        for h in range(2):
            tcp = pltpu.async_copy(
                w_hbm.at[b, ct, :, pl.ds(4 * h, 4), :], table_v, st)
            icp = pltpu.async_copy(idx_src(0), idx_v, si)
            tcp.wait()
            icp.wait()

            def chunk(q, carry0, h=h):
                @pl.when(q > 0)
                def _():
                    pltpu.sync_copy(idx_src(q), idx_v)

                def pair(t, carry, q=q, h=h):
                    for p in range(2):
                        sl2 = 2 * t + p
                        s0 = q * sq + sl2 * 2

                        @pl.when(jnp.logical_or(sl2 >= 2, (q + h) > 0))
                        def _():
                            pltpu.make_async_copy(
                                out_bufs[p],
                                out_hbm.at[b, pl.ds(s0, 2), ct, :,
                                           pl.ds(4 * h, 4), :],
                                sems[p]).wait()

                        @plsc.parallel_loop(0, 32, unroll=4)
                        def ptile(pt2, p=p, sl2=sl2):
                            for gg in range(8):
                                n = idx_v[sl2 * 32 + pt2, pl.ds(gg * 16, 16)]
                                nt = lax.shift_right_logical(n, 7)
                                nj = lax.bitwise_and(n, 127)
                                so = lax.shift_right_logical(pt2, 4)
                                pt = lax.bitwise_and(pt2, 15)
                                for ci_ in range(4):
                                    ci_arr = jnp.full((16,), ci_, jnp.int32)
                                    vals = plsc.load_gather(
                                        table_v, [nt, ci_arr, nj])
                                    out_bufs[p][
                                        so, pt, ci_, pl.ds(gg * 16, 16)
                                    ] = vals
                        pltpu.async_copy(
                            out_bufs[p],
                            out_hbm.at[b, pl.ds(s0, 2), ct, :,
                                       pl.ds(4 * h, 4), :],
                            sems[p])
                    return carry

                lax.fori_loop(0, sq // 4, pair, 0)
                return carry0

            lax.fori_loop(0, nq, chunk, 0)

        for p in range(2):
            pltpu.make_async_copy(
                out_bufs[p],
                out_hbm.at[b, pl.ds(0, 2), ct, :, pl.ds(4, 4), :],
                sems[p]).wait()

    return gather_kernel(w, idx2d)


def _rotl(x, r):
    return (x << jnp.uint32(r)) | (x >> jnp.uint32(32 - r))


def _tf_rounds(x0, x1, rots):
    for r in rots:
        x0 = x0 + x1
        x1 = _rotl(x1, r)
        x1 = x1 ^ x0
    return x0, x1


def _group_indices_2d(b, n):
    key = jax.random.key(42)
    k2 = jax.random.split(key)[1]
    kd = jax.random.key_data(jax.random.split(k2, 2)[1]).astype(jnp.uint32)
    ks0, ks1 = kd[0], kd[1]
    ks2 = ks0 ^ ks1 ^ jnp.uint32(0x1BD11BDA)
    r0 = (13, 15, 26, 6)
    r1 = (17, 29, 16, 24)

    b_ = jnp.arange(b, dtype=jnp.uint32)[:, None, None, None]
    s_ = jnp.arange(_NSAMPLE, dtype=jnp.uint32)[None, :, None, None]
    pt_ = jnp.arange(16, dtype=jnp.uint32)[None, None, :, None]
    pj_ = jnp.arange(128, dtype=jnp.uint32)[None, None, None, :]
    i = (b_ * jnp.uint32(_NPOINTS * _NSAMPLE)
         + (pt_ * jnp.uint32(128) + pj_) * jnp.uint32(_NSAMPLE) + s_)
    i = i.reshape(b * _NSAMPLE * 16, 128)

    x0, x1 = jnp.uint32(0) + ks0, i + ks1
    x0, x1 = _tf_rounds(x0, x1, r0); x0 += ks1; x1 += ks2 + jnp.uint32(1)
    x0, x1 = _tf_rounds(x0, x1, r1); x0 += ks2; x1 += ks0 + jnp.uint32(2)
    x0, x1 = _tf_rounds(x0, x1, r0); x0 += ks0; x1 += ks1 + jnp.uint32(3)
    x0, x1 = _tf_rounds(x0, x1, r1); x0 += ks1; x1 += ks2 + jnp.uint32(4)
    x0, x1 = _tf_rounds(x0, x1, r0); x0 += ks2; x1 += ks0 + jnp.uint32(5)
    return ((x0 ^ x1) & jnp.uint32(n - 1)).astype(jnp.int32)


def kernel(inputs):
    b, n, c = inputs.shape
    idx2d = _group_indices_2d(b, n)
    w = inputs.transpose(0, 2, 1).reshape(b, 2, 8, 128, 128)
    w = w.transpose(0, 1, 3, 2, 4)
    o6 = _sc_gather_tiled(w, idx2d, b, _NSAMPLE)
    out = o6.transpose(0, 1, 2, 4, 3, 5).reshape(b, _NSAMPLE, c, _NPOINTS)
    return out.transpose(0, 3, 1, 2)
